# baseline (speedup 1.0000x reference)
import functools

import jax
import jax.numpy as jnp
from jax import lax
from jax.experimental import pallas as pl
from jax.experimental.pallas import tpu as pltpu
from jax.experimental.pallas import tpu_sc as plsc

N = 10000
E = 320000
D = 128
NC = 2
NS = 16
NW = NC * NS
K = 128
EPW = 10240
NBLK = EPW // K
NPH = 2
PB = NBLK // NPH
NP = 10240
RPT = NP // NS
ZB = RPT // K


def _zero_rows(ref, nrows, width):
    def zrow(i, _):
        for j in range(width // 16):
            ref[i, pl.ds(j * 16, 16)] = jnp.zeros((16,), jnp.float32)
        return 0
    lax.fori_loop(0, nrows, zrow, 0)


def _sc_mesh():
    return plsc.VectorSubcoreMesh(
        core_axis_name="c", subcore_axis_name="s",
        num_cores=NC, num_subcores=NS)


def _sc_layer(with_deg, x_in, ei):
    out_type = [jax.ShapeDtypeStruct((NC, NP, D), jnp.float32)]
    scratch = [
        pltpu.VMEM((2, PB, K), jnp.int32),
        pltpu.VMEM((K, D), jnp.float32),
        pltpu.VMEM((K, D), jnp.float32),
        pltpu.VMEM_SHARED((NP, D), jnp.float32),
        pltpu.SemaphoreType.DMA,
        pltpu.SemaphoreType.DMA,
    ]
    if with_deg:
        out_type.append(jax.ShapeDtypeStruct((NC, NP), jnp.float32))
        scratch += [
            pltpu.VMEM((K,), jnp.float32),
            pltpu.VMEM((RPT,), jnp.float32),
            pltpu.VMEM_SHARED((NP,), jnp.float32),
        ]

    def body(x_hbm, ei_hbm, *rest):
        if with_deg:
            (out_hbm, deg_hbm, idx_v, bufa, bufb, acc_sp, sema, semb,
             ones_v, zer_v, deg_sp) = rest
        else:
            (out_hbm, idx_v, bufa, bufb, acc_sp, sema, semb) = rest
        cid = lax.axis_index("c")
        sid = lax.axis_index("s")
        w = cid * NS + sid

        _zero_rows(bufa, K, D)
        for k in range(ZB):
            pltpu.sync_copy(bufa, acc_sp.at[pl.ds(sid * RPT + k * K, K)])
        if with_deg:
            def zdeg(i, _):
                zer_v[pl.ds(i * 16, 16)] = jnp.zeros((16,), jnp.float32)
                return 0
            lax.fori_loop(0, RPT // 16, zdeg, 0)
            pltpu.sync_copy(zer_v, deg_sp.at[pl.ds(sid * RPT, RPT)])
            def orow(i, _):
                ones_v[pl.ds(i * 16, 16)] = jnp.ones((16,), jnp.float32)
                return 0
            lax.fori_loop(0, K // 16, orow, 0)
        plsc.subcore_barrier()

        def gather(jb, buf, sem):
            pltpu.async_copy(x_hbm.at[idx_v.at[0, jb]], buf, sem)

        def consume(jb, buf, sem):
            pltpu.make_async_copy(x_hbm.at[idx_v.at[0, jb]], buf, sem).wait()
            pltpu.sync_copy(buf, acc_sp.at[idx_v.at[1, jb]], add=True)
            if with_deg:
                pltpu.sync_copy(ones_v, deg_sp.at[idx_v.at[1, jb]], add=True)

        for ph in range(NPH):
            pltpu.sync_copy(ei_hbm.at[w, ph], idx_v)
            gather(0, bufa, sema)
            gather(1, bufb, semb)
            def pbody(p, _):
                j0 = 2 * p
                consume(j0, bufa, sema)
                gather(j0 + 2, bufa, sema)
                consume(j0 + 1, bufb, semb)
                gather(j0 + 3, bufb, semb)
                return 0
            lax.fori_loop(0, PB // 2 - 1, pbody, 0)
            consume(PB - 2, bufa, sema)
            consume(PB - 1, bufb, semb)

        plsc.subcore_barrier()
        pltpu.sync_copy(acc_sp.at[pl.ds(sid * RPT, RPT)],
                        out_hbm.at[cid, pl.ds(sid * RPT, RPT)])
        if with_deg:
            pltpu.sync_copy(deg_sp.at[pl.ds(sid * RPT, RPT)],
                            deg_hbm.at[cid, pl.ds(sid * RPT, RPT)])

    return pl.kernel(body, out_type=tuple(out_type), mesh=_sc_mesh(),
                     scratch_types=tuple(scratch))(x_in, ei)


def _tc_body(p_ref, deg2_ref, x_ref, w_ref, b_ref, o_ref):
    p = p_ref[0] + p_ref[1]
    deg = deg2_ref[:, 0] + deg2_ref[:, 1]
    g = p - deg[:, None] * x_ref[...]
    h = lax.dot_general(g, w_ref[...], (((1,), (1,)), ((), ())),
                        preferred_element_type=jnp.float32,
                        precision=lax.Precision.HIGHEST)
    o_ref[...] = h + deg[:, None] * b_ref[...]


def _tc_layer(P, DEG, xin, W, b, nrows, br):
    return pl.pallas_call(
        _tc_body,
        grid=(nrows // br,),
        in_specs=[
            pl.BlockSpec((NC, br, D), lambda i: (0, i, 0)),
            pl.BlockSpec((br, NC), lambda i: (i, 0)),
            pl.BlockSpec((br, D), lambda i: (i, 0)),
            pl.BlockSpec((D, D), lambda i: (0, 0)),
            pl.BlockSpec((1, D), lambda i: (0, 0)),
        ],
        out_specs=pl.BlockSpec((br, D), lambda i: (i, 0)),
        out_shape=jax.ShapeDtypeStruct((nrows, D), jnp.float32),
    )(P, DEG, xin, W, b.reshape(1, D))


def kernel(x, edge_index, edge_index_inter, W1, b1, W2, b2, W3, b3):
    src = edge_index[0].astype(jnp.int32)
    dst = edge_index[1].astype(jnp.int32)
    pad_n = NW * EPW - E
    fill = jnp.arange(pad_n, dtype=jnp.int32) % (NP - N)
    src_p = jnp.concatenate([src, fill])
    dst_p = jnp.concatenate([dst, N + fill])
    ei = jnp.stack([src_p.reshape(NW, NPH, PB, K),
                    dst_p.reshape(NW, NPH, PB, K)], axis=2)

    P1, DEG = _sc_layer(True, x, ei)
    DEG = DEG.T
    h1 = _tc_layer(P1, DEG, x, W1, b1, N, 2000)
    P2, = _sc_layer(False, h1, ei)
    h2 = _tc_layer(P2, DEG, h1, W2, b2, N, 2000)
    P3, = _sc_layer(False, h2, ei)
    return _tc_layer(P3, DEG, h2, W3, b3, N, 2000)
